# baseline (speedup 1.0000x reference)
import functools

import jax
import jax.numpy as jnp
from jax import lax
from jax.experimental import pallas as pl
from jax.experimental.pallas import tpu as pltpu
from jax.experimental.pallas import tpu_sc as plsc

NUM_EMB = 1000000
DIM = 64
BATCH = 16384

_NC = 2
_NS = 16
_NW = _NC * _NS
_B_PER_W = BATCH // _NW


def _body(w_hbm, idx_hbm, out_hbm, idx_v, rows_v, bf_v, sem):
    wid = lax.axis_index("s") * _NC + lax.axis_index("c")
    base = wid * _B_PER_W

    pltpu.sync_copy(idx_hbm.at[pl.ds(base, _B_PER_W)], idx_v)
    pltpu.async_copy(w_hbm.at[idx_v], rows_v, sem).wait()

    iota = lax.iota(jnp.int32, 16)

    def cast_half_row(i, _):
        r = i // 2
        c = (i % 2) * 32
        rr = jnp.full((16,), r, jnp.int32)
        col = c + 2 * iota
        ev = plsc.load_gather(rows_v, [rr, col])
        od = plsc.load_gather(rows_v, [rr, col + 1])
        bf_v[r, pl.ds(c, 32)] = plsc.pack(
            ev, od, format=plsc.PackFormat.INTERLEAVED)
        return 0

    lax.fori_loop(0, _B_PER_W * 2, cast_half_row, 0)

    pltpu.sync_copy(bf_v, out_hbm.at[pl.ds(base, _B_PER_W)])


_sc_gather_cast = pl.kernel(
    _body,
    out_type=jax.ShapeDtypeStruct((BATCH, DIM), jnp.bfloat16),
    mesh=plsc.VectorSubcoreMesh(
        core_axis_name="c", subcore_axis_name="s",
        num_cores=_NC, num_subcores=_NS),
    compiler_params=pltpu.CompilerParams(
        needs_layout_passes=False, use_tc_tiling_on_sc=False),
    scratch_types=[
        pltpu.VMEM((_B_PER_W,), jnp.int32),
        pltpu.VMEM((_B_PER_W, DIM), jnp.float32),
        pltpu.VMEM((_B_PER_W, DIM), jnp.bfloat16),
        pltpu.SemaphoreType.DMA,
    ],
)


def kernel(weights, inputs, train):
    del train
    return _sc_gather_cast(weights, inputs)
